# Initial kernel scaffold; baseline (speedup 1.0000x reference)
import functools

import jax
import jax.numpy as jnp
from jax import lax
from jax.experimental import pallas as pl
from jax.experimental.pallas import tpu as pltpu
from jax.experimental.pallas import tpu_sc as plsc

N_NODES = 10000
N_PAD = 10240
E = 320000
E_PAD = 327680
NC = 2
NS = 16
NW = NC * NS
CH = 128
NCH = E_PAD // NW // CH
ROWS_PER_TILE = N_PAD // NS


def _make_seg_sum(D):
  mesh = plsc.VectorSubcoreMesh(core_axis_name="c", subcore_axis_name="s")

  @functools.partial(
      pl.kernel,
      mesh=mesh,
      out_type=jax.ShapeDtypeStruct((NC * N_PAD, D), jnp.float32),
      scratch_types=[
          pltpu.VMEM((NCH, CH), jnp.int32),
          pltpu.VMEM((NCH, CH), jnp.int32),
          pltpu.VMEM((CH, D), jnp.float32),
          pltpu.VMEM_SHARED((N_PAD, D), jnp.float32),
          pltpu.SemaphoreType.DMA,
      ],
  )
  def seg_sum(table_hbm, src_hbm, dst_hbm, zeros_hbm, out_hbm,
              src_v, dst_v, rows_v, acc_sh, sem):
    cid = lax.axis_index("c")
    sid = lax.axis_index("s")
    wid = cid * NS + sid
    row0 = sid * ROWS_PER_TILE
    pltpu.sync_copy(zeros_hbm.at[pl.ds(row0, ROWS_PER_TILE)],
                    acc_sh.at[pl.ds(row0, ROWS_PER_TILE)])
    pltpu.sync_copy(src_hbm.at[wid], src_v)
    pltpu.sync_copy(dst_hbm.at[wid], dst_v)
    plsc.subcore_barrier()

    def chunk(c, carry):
      pltpu.async_copy(table_hbm.at[src_v.at[c]], rows_v, sem).wait()
      pltpu.sync_copy(rows_v, acc_sh.at[dst_v.at[c]], add=True)
      return carry

    lax.fori_loop(0, NCH, chunk, 0)
    plsc.subcore_barrier()
    pltpu.sync_copy(acc_sh.at[pl.ds(row0, ROWS_PER_TILE)],
                    out_hbm.at[pl.ds(cid * N_PAD + row0, ROWS_PER_TILE)])

  return seg_sum


_seg_sum_144 = _make_seg_sum(144)
_seg_sum_128 = _make_seg_sum(128)


BR = 1024


def _combine_body(last_layer, a0, a1, c0, c1, xr, wl, bl, wr, out):
  agg = a0[...] + a1[...]
  cnt = c0[...] + c1[...]
  inv = 1.0 / jnp.maximum(cnt, 1.0)
  z = (jnp.dot(agg, wl[...], preferred_element_type=jnp.float32) * inv
       + bl[...]
       + jnp.dot(xr[...], wr[...], preferred_element_type=jnp.float32))
  if last_layer:
    m = jnp.max(z, axis=1, keepdims=True)
    s = jnp.sum(jnp.exp(z - m), axis=1, keepdims=True)
    out[...] = z - m - jnp.log(s)
  else:
    out[...] = jnp.maximum(z, 0.0)


def _make_combine(last_layer):
  grid = (N_PAD // BR,)
  row_blk = pl.BlockSpec((BR, 128), lambda i: (i, 0))
  cnt_blk = pl.BlockSpec((BR, 1), lambda i: (i, 0))
  full_w = pl.BlockSpec((128, 128), lambda i: (0, 0))
  full_b = pl.BlockSpec((1, 128), lambda i: (0, 0))
  return pl.pallas_call(
      functools.partial(_combine_body, last_layer),
      grid=grid,
      in_specs=[row_blk, row_blk, cnt_blk, cnt_blk, row_blk, full_w, full_b,
                full_w],
      out_specs=row_blk,
      out_shape=jax.ShapeDtypeStruct((N_PAD, 128), jnp.float32),
  )


_combine_relu = _make_combine(False)
_combine_lsm = _make_combine(True)


def kernel(x, edge_index, Wl1, bl1, Wr1, Wl2, bl2, Wr2):
  src = edge_index[0].astype(jnp.int32)
  dst = edge_index[1].astype(jnp.int32)
  pad = jnp.full((E_PAD - E,), N_NODES, jnp.int32)
  src_p = jnp.concatenate([src, pad]).reshape(NW, NCH, CH)
  dst_p = jnp.concatenate([dst, pad]).reshape(NW, NCH, CH)

  x_aug = jnp.zeros((N_PAD, 144), jnp.float32)
  x_aug = x_aug.at[:N_NODES, :128].set(x).at[:N_NODES, 128].set(1.0)
  zeros144 = jnp.zeros((N_PAD, 144), jnp.float32)
  zeros128 = jnp.zeros((N_PAD, 128), jnp.float32)

  agg1 = _seg_sum_144(x_aug, src_p, dst_p, zeros144)
  p0, p1 = agg1[:N_PAD], agg1[N_PAD:]
  c0, c1 = p0[:, 128:129], p1[:, 128:129]
  x_pad = x_aug[:, :128]
  h = _combine_relu(p0[:, :128], p1[:, :128], c0, c1, x_pad, Wl1,
                    bl1.reshape(1, 128), Wr1)

  agg2 = _seg_sum_128(h, src_p, dst_p, zeros128)
  out = _combine_lsm(agg2[:N_PAD], agg2[N_PAD:], c0, c1, h, Wl2,
                     bl2.reshape(1, 128), Wr2)
  return out[:N_NODES]


# speedup vs baseline: 3.1420x; 3.1420x over previous
import functools

import jax
import jax.numpy as jnp
from jax import lax
from jax.experimental import pallas as pl
from jax.experimental.pallas import tpu as pltpu
from jax.experimental.pallas import tpu_sc as plsc

N_NODES = 10000
N_PAD = 10240
E = 320000
E_PAD = 327680
NC = 2
NS = 16
NW = NC * NS
CH = 128
NCH = E_PAD // NW // CH
ROWS_PER_TILE = N_PAD // NS


@functools.lru_cache(maxsize=None)
def _make_seg_sum(D):
  mesh = plsc.VectorSubcoreMesh(core_axis_name="c", subcore_axis_name="s")

  @functools.partial(
      pl.kernel,
      mesh=mesh,
      compiler_params=pltpu.CompilerParams(use_tc_tiling_on_sc=False),
      out_type=jax.ShapeDtypeStruct((NC * N_PAD, D), jnp.float32),
      scratch_types=[
          pltpu.VMEM((NCH, CH), jnp.int32),
          pltpu.VMEM((NCH, CH), jnp.int32),
          pltpu.VMEM((CH, D), jnp.float32),
          pltpu.VMEM_SHARED((N_PAD, D), jnp.float32),
          pltpu.SemaphoreType.DMA,
      ],
  )
  def seg_sum(table_hbm, src_hbm, dst_hbm, zeros_hbm, out_hbm,
              src_v, dst_v, rows_v, acc_sh, sem):
    cid = lax.axis_index("c")
    sid = lax.axis_index("s")
    wid = cid * NS + sid
    row0 = sid * ROWS_PER_TILE
    pltpu.sync_copy(zeros_hbm.at[pl.ds(row0, ROWS_PER_TILE)],
                    acc_sh.at[pl.ds(row0, ROWS_PER_TILE)])
    pltpu.sync_copy(src_hbm.at[wid], src_v)
    pltpu.sync_copy(dst_hbm.at[wid], dst_v)
    plsc.subcore_barrier()

    def chunk(c, carry):
      pltpu.async_copy(table_hbm.at[src_v.at[c]], rows_v, sem).wait()
      pltpu.sync_copy(rows_v, acc_sh.at[dst_v.at[c]], add=True)
      return carry

    lax.fori_loop(0, NCH, chunk, 0)
    plsc.subcore_barrier()
    pltpu.sync_copy(acc_sh.at[pl.ds(row0, ROWS_PER_TILE)],
                    out_hbm.at[pl.ds(cid * N_PAD + row0, ROWS_PER_TILE)])

  return seg_sum


BR = 1024


def _combine_body(last_layer, a0, a1, c0, c1, xr, wl, bl, wr, out):
  agg = a0[...] + a1[...]
  cnt = c0[...] + c1[...]
  inv = 1.0 / jnp.maximum(cnt, 1.0)
  z = (jnp.dot(agg, wl[...], preferred_element_type=jnp.float32) * inv
       + bl[...]
       + jnp.dot(xr[...], wr[...], preferred_element_type=jnp.float32))
  if last_layer:
    m = jnp.max(z, axis=1, keepdims=True)
    s = jnp.sum(jnp.exp(z - m), axis=1, keepdims=True)
    out[...] = z - m - jnp.log(s)
  else:
    out[...] = jnp.maximum(z, 0.0)


def _make_combine(last_layer):
  grid = (N_PAD // BR,)
  row_blk = pl.BlockSpec((BR, 128), lambda i: (i, 0))
  cnt_blk = pl.BlockSpec((BR, 1), lambda i: (i, 0))
  full_w = pl.BlockSpec((128, 128), lambda i: (0, 0))
  full_b = pl.BlockSpec((1, 128), lambda i: (0, 0))
  return pl.pallas_call(
      functools.partial(_combine_body, last_layer),
      grid=grid,
      in_specs=[row_blk, row_blk, cnt_blk, cnt_blk, row_blk, full_w, full_b,
                full_w],
      out_specs=row_blk,
      out_shape=jax.ShapeDtypeStruct((N_PAD, 128), jnp.float32),
  )


_combine_relu = _make_combine(False)
_combine_lsm = _make_combine(True)


def kernel(x, edge_index, Wl1, bl1, Wr1, Wl2, bl2, Wr2):
  src = edge_index[0].astype(jnp.int32)
  dst = edge_index[1].astype(jnp.int32)
  pad = jnp.full((E_PAD - E,), N_NODES, jnp.int32)
  src_p = jnp.concatenate([src, pad]).reshape(NW, NCH, CH)
  dst_p = jnp.concatenate([dst, pad]).reshape(NW, NCH, CH)

  x_aug = jnp.zeros((N_PAD, 144), jnp.float32)
  x_aug = x_aug.at[:N_NODES, :128].set(x).at[:N_NODES, 128].set(1.0)
  zeros144 = jnp.zeros((N_PAD, 144), jnp.float32)
  zeros128 = jnp.zeros((N_PAD, 128), jnp.float32)

  agg1 = _make_seg_sum(144)(x_aug, src_p, dst_p, zeros144)
  p0, p1 = agg1[:N_PAD], agg1[N_PAD:]
  c0, c1 = p0[:, 128:129], p1[:, 128:129]
  x_pad = x_aug[:, :128]
  h = _combine_relu(p0[:, :128], p1[:, :128], c0, c1, x_pad, Wl1,
                    bl1.reshape(1, 128), Wr1)

  agg2 = _make_seg_sum(128)(h, src_p, dst_p, zeros128)
  out = _combine_lsm(agg2[:N_PAD], agg2[N_PAD:], c0, c1, h, Wl2,
                     bl2.reshape(1, 128), Wr2)
  return out[:N_NODES]


# speedup vs baseline: 3.4395x; 1.0947x over previous
import functools

import jax
import jax.numpy as jnp
from jax import lax
from jax.experimental import pallas as pl
from jax.experimental.pallas import tpu as pltpu
from jax.experimental.pallas import tpu_sc as plsc

N_NODES = 10000
N_PAD = 10240
E = 320000
E_PAD = 327680
NC = 2
NS = 16
NW = NC * NS
CH = 64
NCH = E_PAD // NW // CH
ROWS_PER_TILE = N_PAD // NS


@functools.lru_cache(maxsize=None)
def _make_seg_sum(D):
  mesh = plsc.VectorSubcoreMesh(core_axis_name="c", subcore_axis_name="s")

  NBUF = 2
  NSLOT = 2 * NBUF
  NIDX = 2 * NSLOT
  PD = 2 * NBUF

  @functools.partial(
      pl.kernel,
      mesh=mesh,
      compiler_params=pltpu.CompilerParams(use_tc_tiling_on_sc=False),
      out_type=jax.ShapeDtypeStruct((NC * N_PAD, D), jnp.float32),
      scratch_types=[
          pltpu.VMEM((NIDX, 2, CH), jnp.int32),
          pltpu.VMEM((NSLOT, CH, D), jnp.float32),
          pltpu.VMEM_SHARED((N_PAD, D), jnp.float32),
      ] + [pltpu.SemaphoreType.DMA] * (NIDX + 2 * NSLOT),
  )
  def seg_sum(table_hbm, edge_hbm, zeros_hbm, out_hbm,
              idx_v, rows_v, acc_sh, *sems):
    isems = sems[:NIDX]
    gsems = sems[NIDX:NIDX + NSLOT]
    ssems = sems[NIDX + NSLOT:]
    cid = lax.axis_index("c")
    sid = lax.axis_index("s")
    wid = cid * NS + sid
    row0 = sid * ROWS_PER_TILE
    pltpu.sync_copy(zeros_hbm.at[pl.ds(row0, ROWS_PER_TILE)],
                    acc_sh.at[pl.ds(row0, ROWS_PER_TILE)])
    plsc.subcore_barrier()

    def idx_fetch(c, j):
      pltpu.async_copy(edge_hbm.at[wid, c], idx_v.at[j], isems[j])

    def idx_wait(c, j):
      pltpu.make_async_copy(edge_hbm.at[wid, c], idx_v.at[j],
                            isems[j]).wait()

    def gather_start(c, j):
      k = j % NSLOT
      pltpu.async_copy(table_hbm.at[idx_v.at[j, 0]], rows_v.at[k],
                       gsems[k])

    def gather_wait(j):
      k = j % NSLOT
      pltpu.make_async_copy(table_hbm.at[idx_v.at[j, 0]],
                            rows_v.at[k], gsems[k]).wait()

    def scatter_start(j):
      k = j % NSLOT
      pltpu.async_copy(rows_v.at[k], acc_sh.at[idx_v.at[j, 1]],
                       ssems[k], add=True)

    def scatter_wait(j):
      k = j % NSLOT
      pltpu.make_async_copy(rows_v.at[k], acc_sh.at[idx_v.at[j, 1]],
                            ssems[k]).wait()

    def step(c, j, first, last_fetch, last_issue):
      if not last_fetch:
        idx_fetch(c + PD, (j + PD) % NIDX)
      gather_wait(j)
      scatter_start(j)
      if not first:
        scatter_wait((j + NBUF) % NIDX)
      if not last_issue:
        idx_wait(c + NBUF, (j + NBUF) % NIDX)
        gather_start(c + NBUF, (j + NBUF) % NIDX)

    for c in range(PD):
      idx_fetch(c, c)
    for c in range(NBUF):
      idx_wait(c, c)
      gather_start(c, c)
    for j in range(NIDX):
      step(j, j, first=(j < NBUF), last_fetch=False, last_issue=False)

    def group(g, carry):
      for j in range(NIDX):
        step(g * NIDX + j, j, False, False, False)
      return carry

    lax.fori_loop(1, NCH // NIDX - 1, group, 0)
    for j in range(NIDX):
      c = (NCH // NIDX - 1) * NIDX + j
      step(c, j, False, last_fetch=(c + PD >= NCH),
           last_issue=(c + NBUF >= NCH))
    for c in range(NCH - NBUF, NCH):
      scatter_wait(c % NIDX)

    plsc.subcore_barrier()
    pltpu.sync_copy(acc_sh.at[pl.ds(row0, ROWS_PER_TILE)],
                    out_hbm.at[pl.ds(cid * N_PAD + row0, ROWS_PER_TILE)])

  return seg_sum


BR = 1024


def _combine_body(last_layer, a0, a1, c0, c1, xr, wl, bl, wr, out):
  agg = a0[...] + a1[...]
  cnt = c0[...] + c1[...]
  inv = 1.0 / jnp.maximum(cnt, 1.0)
  z = (jnp.dot(agg, wl[...], preferred_element_type=jnp.float32) * inv
       + bl[...]
       + jnp.dot(xr[...], wr[...], preferred_element_type=jnp.float32))
  if last_layer:
    m = jnp.max(z, axis=1, keepdims=True)
    s = jnp.sum(jnp.exp(z - m), axis=1, keepdims=True)
    out[...] = z - m - jnp.log(s)
  else:
    out[...] = jnp.maximum(z, 0.0)


def _make_combine(last_layer):
  grid = (N_PAD // BR,)
  row_blk = pl.BlockSpec((BR, 128), lambda i: (i, 0))
  cnt_blk = pl.BlockSpec((BR, 1), lambda i: (i, 0))
  full_w = pl.BlockSpec((128, 128), lambda i: (0, 0))
  full_b = pl.BlockSpec((1, 128), lambda i: (0, 0))
  return pl.pallas_call(
      functools.partial(_combine_body, last_layer),
      grid=grid,
      in_specs=[row_blk, row_blk, cnt_blk, cnt_blk, row_blk, full_w, full_b,
                full_w],
      out_specs=row_blk,
      out_shape=jax.ShapeDtypeStruct((N_PAD, 128), jnp.float32),
  )


_combine_relu = _make_combine(False)
_combine_lsm = _make_combine(True)


def kernel(x, edge_index, Wl1, bl1, Wr1, Wl2, bl2, Wr2):
  src = edge_index[0].astype(jnp.int32)
  dst = edge_index[1].astype(jnp.int32)
  pad = jnp.full((E_PAD - E,), N_NODES, jnp.int32)
  src_p = jnp.concatenate([src, pad]).reshape(NW, NCH, CH)
  dst_p = jnp.concatenate([dst, pad]).reshape(NW, NCH, CH)
  edge_p = jnp.stack([src_p, dst_p], axis=2)

  x_aug = jnp.zeros((N_PAD, 144), jnp.float32)
  x_aug = x_aug.at[:N_NODES, :128].set(x).at[:N_NODES, 128].set(1.0)
  zeros144 = jnp.zeros((N_PAD, 144), jnp.float32)
  zeros128 = jnp.zeros((N_PAD, 128), jnp.float32)

  agg1 = _make_seg_sum(144)(x_aug, edge_p, zeros144)
  p0, p1 = agg1[:N_PAD], agg1[N_PAD:]
  c0, c1 = p0[:, 128:129], p1[:, 128:129]
  x_pad = x_aug[:, :128]
  h = _combine_relu(p0[:, :128], p1[:, :128], c0, c1, x_pad, Wl1,
                    bl1.reshape(1, 128), Wr1)

  agg2 = _make_seg_sum(128)(h, edge_p, zeros128)
  out = _combine_lsm(agg2[:N_PAD], agg2[N_PAD:], c0, c1, h, Wl2,
                     bl2.reshape(1, 128), Wr2)
  return out[:N_NODES]
